# Initial kernel scaffold; baseline (speedup 1.0000x reference)
import jax
import jax.numpy as jnp
from jax.experimental import pallas as pl


def kernel(vertex_attr, edgeij_pair, edge_attr, g, batch):
    raise NotImplementedError("write your pallas kernel here")


# speedup vs baseline: 1.5929x; 1.5929x over previous
import functools

import jax
import jax.numpy as jnp
from jax import lax
from jax.experimental import pallas as pl
from jax.experimental.pallas import tpu as pltpu
from jax.experimental.pallas import tpu_sc as plsc

N = 100000
E = 3200000
D_E = 4
DP = 8

L = 128
ROWS = E // L
NC = 2
NS = 16
NW = NC * NS
GROUPS = ROWS // 8
GB = GROUPS // NW
EXTRA = GROUPS - GB * NW
CHG = 2
CH = CHG * 8
FULL = GB // CHG
TAIL = GB - FULL * CHG
NP = 100096
VS = NP // NS


def _sc_scatter_partials(dst2d, edge_attr, zeros):
    mesh = plsc.VectorSubcoreMesh(core_axis_name="c", subcore_axis_name="s")

    @functools.partial(
        pl.kernel,
        out_type=jax.ShapeDtypeStruct((NC * NP, DP), jnp.float32),
        mesh=mesh,
        compiler_params=pltpu.CompilerParams(use_tc_tiling_on_sc=False),
        scratch_types=[
            pltpu.VMEM((CH, L), jnp.int32),
            pltpu.VMEM((CH * L, DP), jnp.float32),
            pltpu.VMEM_SHARED((NP, DP), jnp.float32),
        ],
    )
    def run(dst_hbm, edge_hbm, zero_hbm, part_hbm, ibuf, ebuf, acc):
        c = lax.axis_index("c")
        s = lax.axis_index("s")
        wid = s * NC + c
        start = (wid * GB + jnp.minimum(wid, EXTRA)) * 8

        pltpu.sync_copy(zero_hbm.at[pl.ds(s * VS, VS), :],
                        acc.at[pl.ds(s * VS, VS), :])
        plsc.subcore_barrier()

        def chunk(row0, nrows):
            pltpu.sync_copy(dst_hbm.at[pl.ds(row0, nrows), :],
                            ibuf.at[pl.ds(0, nrows), :])
            pltpu.sync_copy(edge_hbm.at[pl.ds(row0 * L, nrows * L), :],
                            ebuf.at[pl.ds(0, nrows * L), pl.ds(0, D_E)])
            for j in range(nrows):
                pltpu.sync_copy(ebuf.at[pl.ds(j * L, L), :],
                                acc.at[ibuf.at[j]], add=True)

        def body(g, carry):
            chunk(start + g * CH, CH)
            return carry

        lax.fori_loop(0, FULL, body, 0)
        chunk(start + FULL * CH, TAIL * 8)

        @pl.when(wid < EXTRA)
        def _():
            chunk(start + GB * 8, 8)

        plsc.subcore_barrier()
        pltpu.sync_copy(acc.at[pl.ds(s * VS, VS), :],
                        part_hbm.at[pl.ds(c * NP + s * VS, VS), :])

    return run(dst2d, edge_attr, zeros)


BR = 2000


def _combine_body(p_ref, b_ref, o_ref):
    p = p_ref[0, :, :D_E] + p_ref[1, :, :D_E]
    b = b_ref[...]
    o_ref[...] = jnp.concatenate([b, b * p], axis=1)


def _tc_combine(partials, b):
    return pl.pallas_call(
        _combine_body,
        grid=(N // BR,),
        in_specs=[
            pl.BlockSpec((2, BR, DP), lambda i: (0, i, 0)),
            pl.BlockSpec((BR, 1), lambda i: (i, 0)),
        ],
        out_specs=pl.BlockSpec((BR, 1 + D_E), lambda i: (i, 0)),
        out_shape=jax.ShapeDtypeStruct((N, 1 + D_E), jnp.float32),
    )(partials, b)


def kernel(vertex_attr, edgeij_pair, edge_attr, g, batch):
    dst2d = edgeij_pair[1].reshape(ROWS, L)
    zeros = jnp.zeros((NP, DP), jnp.float32)
    part = _sc_scatter_partials(dst2d, edge_attr, zeros)
    b = vertex_attr[:, :1]
    return _tc_combine(part.reshape(NC, NP, DP), b)


# speedup vs baseline: 1.6195x; 1.0168x over previous
import functools

import jax
import jax.numpy as jnp
from jax import lax
from jax.experimental import pallas as pl
from jax.experimental.pallas import tpu as pltpu
from jax.experimental.pallas import tpu_sc as plsc

N = 100000
E = 3200000
D_E = 4
DP = 8

L = 128
ROWS = E // L
NC = 2
NS = 16
NW = NC * NS
PR = 784
CH = 16
NCH = PR // CH
LAST_ROWS = ROWS - (NW - 1) * PR
LAST_NCH = LAST_ROWS // CH
NP = 100096
VS = NP // NS


def _sc_scatter_partials(dst2d, edge_attr, zeros):
    mesh = plsc.VectorSubcoreMesh(core_axis_name="c", subcore_axis_name="s")

    @functools.partial(
        pl.kernel,
        out_type=jax.ShapeDtypeStruct((NC * NP, DP), jnp.float32),
        mesh=mesh,
        compiler_params=pltpu.CompilerParams(use_tc_tiling_on_sc=False),
        scratch_types=[
            pltpu.VMEM((CH, L), jnp.int32),
            pltpu.VMEM((CH, L), jnp.int32),
            pltpu.VMEM((CH * L, DP), jnp.float32),
            pltpu.VMEM((CH * L, DP), jnp.float32),
            pltpu.VMEM_SHARED((NP, DP), jnp.float32),
            pltpu.SemaphoreType.DMA,
            pltpu.SemaphoreType.DMA,
            pltpu.SemaphoreType.DMA,
        ],
    )
    def run(dst_hbm, edge_hbm, zero_hbm, part_hbm,
            ibuf0, ibuf1, ebuf0, ebuf1, acc, ssem0, ssem1, csem):
        c = lax.axis_index("c")
        s = lax.axis_index("s")
        wid = s * NC + c
        start = wid * PR
        nfull = jnp.where(wid == NW - 1, LAST_NCH, NCH)
        npairs = (nfull - 1) // 2

        def stage_parts(g, ibuf, ebuf, ssem):
            row0 = start + g * CH
            i = pltpu.make_async_copy(dst_hbm.at[pl.ds(row0, CH), :],
                                      ibuf, ssem)
            e = pltpu.make_async_copy(edge_hbm.at[pl.ds(row0 * L, CH * L), :],
                                      ebuf.at[:, pl.ds(0, D_E)], ssem)
            return i, e

        def stage(g, ibuf, ebuf, ssem):
            i, e = stage_parts(g, ibuf, ebuf, ssem)
            i.start()
            e.start()

        def wait_stage(g, ibuf, ebuf, ssem):
            i, e = stage_parts(g, ibuf, ebuf, ssem)
            i.wait()
            e.wait()

        def scatter(ibuf, ebuf):
            hs = [pltpu.async_copy(ebuf.at[pl.ds(j * L, L), :],
                                   acc.at[ibuf.at[j]], csem, add=True)
                  for j in range(CH)]
            return hs

        def drain(hs):
            for h in hs:
                h.wait()

        stage(0, ibuf0, ebuf0, ssem0)
        pltpu.sync_copy(zero_hbm.at[pl.ds(s * VS, VS), :],
                        acc.at[pl.ds(s * VS, VS), :])
        plsc.subcore_barrier()

        def body(k, carry):
            g = 2 * k
            wait_stage(g, ibuf0, ebuf0, ssem0)
            stage(g + 1, ibuf1, ebuf1, ssem1)
            drain(scatter(ibuf0, ebuf0))
            wait_stage(g + 1, ibuf1, ebuf1, ssem1)
            stage(g + 2, ibuf0, ebuf0, ssem0)
            drain(scatter(ibuf1, ebuf1))
            return carry

        lax.fori_loop(0, npairs, body, 0)

        wait_stage(nfull - 1, ibuf0, ebuf0, ssem0)
        drain(scatter(ibuf0, ebuf0))

        @pl.when(wid == NW - 1)
        def _():
            row0 = start + LAST_NCH * CH
            pltpu.sync_copy(dst_hbm.at[pl.ds(row0, 8), :],
                            ibuf1.at[pl.ds(0, 8), :])
            pltpu.sync_copy(edge_hbm.at[pl.ds(row0 * L, 8 * L), :],
                            ebuf1.at[pl.ds(0, 8 * L), pl.ds(0, D_E)])
            for j in range(8):
                pltpu.sync_copy(ebuf1.at[pl.ds(j * L, L), :],
                                acc.at[ibuf1.at[j]], add=True)

        plsc.subcore_barrier()
        pltpu.sync_copy(acc.at[pl.ds(s * VS, VS), :],
                        part_hbm.at[pl.ds(c * NP + s * VS, VS), :])

    return run(dst2d, edge_attr, zeros)


BR = 2000


def _combine_body(p_ref, b_ref, o_ref):
    p = p_ref[0, :, :D_E] + p_ref[1, :, :D_E]
    b = b_ref[...]
    o_ref[...] = jnp.concatenate([b, b * p], axis=1)


def _tc_combine(partials, b):
    return pl.pallas_call(
        _combine_body,
        grid=(N // BR,),
        in_specs=[
            pl.BlockSpec((2, BR, DP), lambda i: (0, i, 0)),
            pl.BlockSpec((BR, 1), lambda i: (i, 0)),
        ],
        out_specs=pl.BlockSpec((BR, 1 + D_E), lambda i: (i, 0)),
        out_shape=jax.ShapeDtypeStruct((N, 1 + D_E), jnp.float32),
    )(partials, b)


def kernel(vertex_attr, edgeij_pair, edge_attr, g, batch):
    dst2d = edgeij_pair[1].reshape(ROWS, L)
    zeros = jnp.zeros((NP, DP), jnp.float32)
    part = _sc_scatter_partials(dst2d, edge_attr, zeros)
    b = vertex_attr[:, :1]
    return _tc_combine(part.reshape(NC, NP, DP), b)


# speedup vs baseline: 28.9817x; 17.8949x over previous
import functools

import jax
import jax.numpy as jnp
from jax import lax
from jax.experimental import pallas as pl
from jax.experimental.pallas import tpu as pltpu
from jax.experimental.pallas import tpu_sc as plsc

N = 100000
E = 3200000
D_E = 4
DP = 8

L = 128
ROWS = E // L
NC = 2
NS = 16
NW = NC * NS
PR = 784
CH = 16
NCH = PR // CH
LAST_ROWS = ROWS - (NW - 1) * PR
LAST_NCH = LAST_ROWS // CH
NP = 100096
VS = NP // NS


def _sc_scatter_partials(dst2d, eblk, zeros):
    mesh = plsc.VectorSubcoreMesh(core_axis_name="c", subcore_axis_name="s")

    @functools.partial(
        pl.kernel,
        out_type=jax.ShapeDtypeStruct((NC * NP, DP), jnp.float32),
        mesh=mesh,
        compiler_params=pltpu.CompilerParams(use_tc_tiling_on_sc=False,
                                             needs_layout_passes=False),
        scratch_types=[
            pltpu.VMEM((CH, L), jnp.int32),
            pltpu.VMEM((CH, L), jnp.int32),
            pltpu.VMEM((CH, D_E, L), jnp.float32),
            pltpu.VMEM((CH, D_E, L), jnp.float32),
            pltpu.VMEM((CH * L, DP), jnp.float32),
            pltpu.VMEM((CH * L, DP), jnp.float32),
            pltpu.VMEM_SHARED((NP, DP), jnp.float32),
            pltpu.SemaphoreType.DMA,
            pltpu.SemaphoreType.DMA,
            pltpu.SemaphoreType.DMA,
        ],
    )
    def run(dst_hbm, eblk_hbm, zero_hbm, part_hbm,
            ibuf0, ibuf1, pbuf0, pbuf1, ebuf0, ebuf1,
            acc, ssem0, ssem1, csem):
        c = lax.axis_index("c")
        s = lax.axis_index("s")
        wid = s * NC + c
        start = wid * PR
        nfull = jnp.where(wid == NW - 1, LAST_NCH, NCH)
        npairs = (nfull - 1) // 2
        lanes = lax.iota(jnp.int32, 16)

        def stage_parts(g, ibuf, pbuf, ssem):
            row0 = start + g * CH
            i = pltpu.make_async_copy(dst_hbm.at[pl.ds(row0, CH), :],
                                      ibuf, ssem)
            e = pltpu.make_async_copy(eblk_hbm.at[pl.ds(row0, CH), :, :],
                                      pbuf, ssem)
            return i, e

        def stage(g, ibuf, pbuf, ssem):
            i, e = stage_parts(g, ibuf, pbuf, ssem)
            i.start()
            e.start()

        def wait_stage(g, ibuf, pbuf, ssem):
            i, e = stage_parts(g, ibuf, pbuf, ssem)
            i.wait()
            e.wait()

        def widen(pbuf, ebuf, nb):
            def wb(b, carry):
                pb = pbuf.at[b]
                eb = ebuf.at[pl.ds(b * L, L), :]
                for f in range(D_E):
                    colv = jnp.full((16,), f, jnp.int32)
                    for i0 in range(0, L, 16):
                        x = pb[f, pl.ds(i0, 16)]
                        plsc.store_scatter(eb, [lanes + i0, colv], x)
                return carry
            lax.fori_loop(0, nb, wb, 0)

        def scatter(ibuf, ebuf):
            return [pltpu.async_copy(ebuf.at[pl.ds(j * L, L), :],
                                     acc.at[ibuf.at[j]], csem, add=True)
                    for j in range(CH)]

        def drain(hs):
            for h in hs:
                h.wait()

        stage(0, ibuf0, pbuf0, ssem0)
        pltpu.sync_copy(zero_hbm.at[pl.ds(s * VS, VS), :],
                        acc.at[pl.ds(s * VS, VS), :])
        plsc.subcore_barrier()

        def body(k, carry):
            g = 2 * k
            wait_stage(g, ibuf0, pbuf0, ssem0)
            stage(g + 1, ibuf1, pbuf1, ssem1)
            widen(pbuf0, ebuf0, CH)
            h0 = scatter(ibuf0, ebuf0)
            wait_stage(g + 1, ibuf1, pbuf1, ssem1)
            widen(pbuf1, ebuf1, CH)
            h1 = scatter(ibuf1, ebuf1)
            drain(h0)
            stage(g + 2, ibuf0, pbuf0, ssem0)
            drain(h1)
            return carry

        lax.fori_loop(0, npairs, body, 0)

        wait_stage(nfull - 1, ibuf0, pbuf0, ssem0)
        widen(pbuf0, ebuf0, CH)
        drain(scatter(ibuf0, ebuf0))

        @pl.when(wid == NW - 1)
        def _():
            row0 = start + LAST_NCH * CH
            pltpu.sync_copy(dst_hbm.at[pl.ds(row0, 8), :],
                            ibuf1.at[pl.ds(0, 8), :])
            pltpu.sync_copy(eblk_hbm.at[pl.ds(row0, 8), :, :],
                            pbuf1.at[pl.ds(0, 8), :, :])
            widen(pbuf1, ebuf1, 8)
            for j in range(8):
                pltpu.sync_copy(ebuf1.at[pl.ds(j * L, L), :],
                                acc.at[ibuf1.at[j]], add=True)

        plsc.subcore_barrier()
        pltpu.sync_copy(acc.at[pl.ds(s * VS, VS), :],
                        part_hbm.at[pl.ds(c * NP + s * VS, VS), :])

    return run(dst2d, eblk, zeros)


BR = 2000


def _combine_body(p_ref, b_ref, o_ref):
    p = p_ref[0, :, :D_E] + p_ref[1, :, :D_E]
    b = b_ref[...]
    o_ref[...] = jnp.concatenate([b, b * p], axis=1)


def _tc_combine(partials, b):
    return pl.pallas_call(
        _combine_body,
        grid=(N // BR,),
        in_specs=[
            pl.BlockSpec((2, BR, DP), lambda i: (0, i, 0)),
            pl.BlockSpec((BR, 1), lambda i: (i, 0)),
        ],
        out_specs=pl.BlockSpec((BR, 1 + D_E), lambda i: (i, 0)),
        out_shape=jax.ShapeDtypeStruct((N, 1 + D_E), jnp.float32),
    )(partials, b)


def kernel(vertex_attr, edgeij_pair, edge_attr, g, batch):
    dst2d = edgeij_pair[1].reshape(ROWS, L)
    eblk = edge_attr.reshape(ROWS, L, D_E).transpose(0, 2, 1)
    zeros = jnp.zeros((NP, DP), jnp.float32)
    part = _sc_scatter_partials(dst2d, eblk, zeros)
    b = vertex_attr[:, :1]
    return _tc_combine(part.reshape(NC, NP, DP), b)


# speedup vs baseline: 36.5675x; 1.2617x over previous
import functools

import jax
import jax.numpy as jnp
from jax import lax
from jax.experimental import pallas as pl
from jax.experimental.pallas import tpu as pltpu
from jax.experimental.pallas import tpu_sc as plsc

N = 100000
E = 3200000
D_E = 4
DP = 8

L = 128
ROWS = E // L
NC = 2
NS = 16
NW = NC * NS
PR = 784
CH = 16
NCH = PR // CH
LAST_ROWS = ROWS - (NW - 1) * PR
LAST_NCH = LAST_ROWS // CH
NP = 100096
VS = NP // NS


def _sc_scatter_partials(dst2d, eblk, zeros):
    mesh = plsc.VectorSubcoreMesh(core_axis_name="c", subcore_axis_name="s")

    @functools.partial(
        pl.kernel,
        out_type=jax.ShapeDtypeStruct((NC * NP, DP), jnp.float32),
        mesh=mesh,
        compiler_params=pltpu.CompilerParams(use_tc_tiling_on_sc=False,
                                             needs_layout_passes=False),
        scratch_types=[
            pltpu.VMEM((CH, L), jnp.int32),
            pltpu.VMEM((CH, L), jnp.int32),
            pltpu.VMEM((CH, D_E, L), jnp.float32),
            pltpu.VMEM((CH, D_E, L), jnp.float32),
            pltpu.VMEM((CH * L, DP), jnp.float32),
            pltpu.VMEM((CH * L, DP), jnp.float32),
            pltpu.VMEM_SHARED((NP, DP), jnp.float32),
            pltpu.SemaphoreType.DMA,
            pltpu.SemaphoreType.DMA,
            pltpu.SemaphoreType.DMA,
        ],
    )
    def run(dst_hbm, eblk_hbm, zero_hbm, part_hbm,
            ibuf0, ibuf1, pbuf0, pbuf1, ebuf0, ebuf1,
            acc, ssem0, ssem1, csem):
        c = lax.axis_index("c")
        s = lax.axis_index("s")
        wid = s * NC + c
        start = wid * PR
        nfull = jnp.where(wid == NW - 1, LAST_NCH, NCH)
        npairs = (nfull - 1) // 2
        lanes = lax.iota(jnp.int32, 16)

        def stage_parts(g, ibuf, pbuf, ssem):
            row0 = start + g * CH
            i = pltpu.make_async_copy(dst_hbm.at[pl.ds(row0, CH), :],
                                      ibuf, ssem)
            e = pltpu.make_async_copy(eblk_hbm.at[pl.ds(row0, CH), :, :],
                                      pbuf, ssem)
            return i, e

        def stage(g, ibuf, pbuf, ssem):
            i, e = stage_parts(g, ibuf, pbuf, ssem)
            i.start()
            e.start()

        def wait_stage(g, ibuf, pbuf, ssem):
            i, e = stage_parts(g, ibuf, pbuf, ssem)
            i.wait()
            e.wait()

        def widen(pbuf, ebuf, nb):
            def wb(b, carry):
                pb = pbuf.at[b]
                eb = ebuf.at[pl.ds(b * L, L), :]
                for f in range(D_E):
                    colv = jnp.full((16,), f, jnp.int32)
                    for i0 in range(0, L, 16):
                        x = pb[f, pl.ds(i0, 16)]
                        plsc.store_scatter(eb, [lanes + i0, colv], x)
                return carry
            lax.fori_loop(0, nb, wb, 0)

        def scatter(ibuf, ebuf):
            return [pltpu.async_copy(ebuf.at[pl.ds(j * L, L), :],
                                     acc.at[ibuf.at[j]], csem, add=True)
                    for j in range(CH)]

        def drain(hs):
            for h in hs:
                h.wait()

        stage(0, ibuf0, pbuf0, ssem0)
        pltpu.sync_copy(zero_hbm.at[pl.ds(s * VS, VS), :],
                        acc.at[pl.ds(s * VS, VS), :])
        plsc.subcore_barrier()

        def body(k, carry):
            g = 2 * k
            wait_stage(g, ibuf0, pbuf0, ssem0)
            stage(g + 1, ibuf1, pbuf1, ssem1)
            widen(pbuf0, ebuf0, CH)
            h0 = scatter(ibuf0, ebuf0)
            wait_stage(g + 1, ibuf1, pbuf1, ssem1)
            widen(pbuf1, ebuf1, CH)
            h1 = scatter(ibuf1, ebuf1)
            drain(h0)
            stage(g + 2, ibuf0, pbuf0, ssem0)
            drain(h1)
            return carry

        lax.fori_loop(0, npairs, body, 0)

        wait_stage(nfull - 1, ibuf0, pbuf0, ssem0)
        widen(pbuf0, ebuf0, CH)
        drain(scatter(ibuf0, ebuf0))

        @pl.when(wid == NW - 1)
        def _():
            row0 = start + LAST_NCH * CH
            pltpu.sync_copy(dst_hbm.at[pl.ds(row0, 8), :],
                            ibuf1.at[pl.ds(0, 8), :])
            pltpu.sync_copy(eblk_hbm.at[pl.ds(row0, 8), :, :],
                            pbuf1.at[pl.ds(0, 8), :, :])
            widen(pbuf1, ebuf1, 8)
            for j in range(8):
                pltpu.sync_copy(ebuf1.at[pl.ds(j * L, L), :],
                                acc.at[ibuf1.at[j]], add=True)

        plsc.subcore_barrier()
        pltpu.sync_copy(acc.at[pl.ds(s * VS, VS), :],
                        part_hbm.at[pl.ds(c * NP + s * VS, VS), :])

    return run(dst2d, eblk, zeros)


PW = NP * DP // 128
BR = PW // 2


def _combine_body(p_ref, b_ref, o_ref):
    q = p_ref[0] + p_ref[1]
    b = b_ref[...]
    y = pltpu.roll(q * b, 1, 1)
    l = lax.broadcasted_iota(jnp.int32, (BR, 128), 1) % DP
    o_ref[...] = jnp.where(l == 0, b, jnp.where(l <= D_E, y, 0.0))


def _tc_combine(partials, bx):
    return pl.pallas_call(
        _combine_body,
        grid=(2,),
        in_specs=[
            pl.BlockSpec((2, BR, 128), lambda i: (0, i, 0)),
            pl.BlockSpec((BR, 128), lambda i: (i, 0)),
        ],
        out_specs=pl.BlockSpec((BR, 128), lambda i: (i, 0)),
        out_shape=jax.ShapeDtypeStruct((PW, 128), jnp.float32),
    )(partials, bx)


def kernel(vertex_attr, edgeij_pair, edge_attr, g, batch):
    dst2d = edgeij_pair[1].reshape(ROWS, L)
    eblk = edge_attr.reshape(ROWS, L, D_E).transpose(0, 2, 1)
    zeros = jnp.zeros((NP, DP), jnp.float32)
    part = _sc_scatter_partials(dst2d, eblk, zeros)
    p3 = part.reshape(NC, PW, 128)
    bpad = jnp.pad(vertex_attr[:, :1], ((0, NP - N), (0, 0)))
    bx = jnp.broadcast_to(bpad, (NP, DP)).reshape(PW, 128)
    wide = _tc_combine(p3, bx)
    return wide.reshape(NP, DP)[:N, :1 + D_E]


# speedup vs baseline: 49.8849x; 1.3642x over previous
import functools

import jax
import jax.numpy as jnp
from jax import lax
from jax.experimental import pallas as pl
from jax.experimental.pallas import tpu as pltpu
from jax.experimental.pallas import tpu_sc as plsc

N = 100000
E = 3200000
D_E = 4
DP = 8

L = 128
ROWS = E // L
NC = 2
NS = 16
NW = NC * NS
PR = 784
CH = 16
NCH = PR // CH
LAST_ROWS = ROWS - (NW - 1) * PR
LAST_NCH = LAST_ROWS // CH
NP = 100096
VS = NP // NS


def _sc_scatter_partials(dst2d, eblk, zeros):
    mesh = plsc.VectorSubcoreMesh(core_axis_name="c", subcore_axis_name="s")

    @functools.partial(
        pl.kernel,
        out_type=jax.ShapeDtypeStruct((NC * NP, DP), jnp.float32),
        mesh=mesh,
        compiler_params=pltpu.CompilerParams(use_tc_tiling_on_sc=False,
                                             needs_layout_passes=False),
        scratch_types=[
            pltpu.VMEM((CH, 1, L), jnp.int32),
            pltpu.VMEM((CH, 1, L), jnp.int32),
            pltpu.VMEM((CH, D_E, L), jnp.float32),
            pltpu.VMEM((CH, D_E, L), jnp.float32),
            pltpu.VMEM((CH * L, DP), jnp.float32),
            pltpu.VMEM((CH * L, DP), jnp.float32),
            pltpu.VMEM_SHARED((NP, DP), jnp.float32),
            pltpu.SemaphoreType.DMA,
            pltpu.SemaphoreType.DMA,
            pltpu.SemaphoreType.DMA,
        ],
    )
    def run(dst_hbm, eblk_hbm, zero_hbm, part_hbm,
            ibuf0, ibuf1, pbuf0, pbuf1, ebuf0, ebuf1,
            acc, ssem0, ssem1, csem):
        c = lax.axis_index("c")
        s = lax.axis_index("s")
        wid = s * NC + c
        start = wid * PR
        nfull = jnp.where(wid == NW - 1, LAST_NCH, NCH)
        npairs = (nfull - 1) // 2
        lanes = lax.iota(jnp.int32, 16)

        def stage_parts(g, ibuf, pbuf, ssem):
            row0 = start + g * CH
            i = pltpu.make_async_copy(dst_hbm.at[pl.ds(row0, CH),
                                                 pl.ds(1, 1), :],
                                      ibuf, ssem)
            e = pltpu.make_async_copy(eblk_hbm.at[pl.ds(row0, CH), :, :],
                                      pbuf, ssem)
            return i, e

        def stage(g, ibuf, pbuf, ssem):
            i, e = stage_parts(g, ibuf, pbuf, ssem)
            i.start()
            e.start()

        def wait_stage(g, ibuf, pbuf, ssem):
            i, e = stage_parts(g, ibuf, pbuf, ssem)
            i.wait()
            e.wait()

        def widen(pbuf, ebuf, nb):
            def wb(b, carry):
                pb = pbuf.at[b]
                eb = ebuf.at[pl.ds(b * L, L), :]
                for f in range(D_E):
                    colv = jnp.full((16,), f, jnp.int32)
                    for i0 in range(0, L, 16):
                        x = pb[f, pl.ds(i0, 16)]
                        plsc.store_scatter(eb, [lanes + i0, colv], x)
                return carry
            lax.fori_loop(0, nb, wb, 0)

        def scatter(ibuf, ebuf):
            return [pltpu.async_copy(ebuf.at[pl.ds(j * L, L), :],
                                     acc.at[ibuf.at[j, 0]], csem, add=True)
                    for j in range(CH)]

        def drain(hs):
            for h in hs:
                h.wait()

        stage(0, ibuf0, pbuf0, ssem0)
        pltpu.sync_copy(zero_hbm.at[pl.ds(s * VS, VS), :],
                        acc.at[pl.ds(s * VS, VS), :])
        plsc.subcore_barrier()

        def body(k, carry):
            g = 2 * k
            wait_stage(g, ibuf0, pbuf0, ssem0)
            stage(g + 1, ibuf1, pbuf1, ssem1)
            widen(pbuf0, ebuf0, CH)
            h0 = scatter(ibuf0, ebuf0)
            wait_stage(g + 1, ibuf1, pbuf1, ssem1)
            widen(pbuf1, ebuf1, CH)
            h1 = scatter(ibuf1, ebuf1)
            drain(h0)
            stage(g + 2, ibuf0, pbuf0, ssem0)
            drain(h1)
            return carry

        lax.fori_loop(0, npairs, body, 0)

        wait_stage(nfull - 1, ibuf0, pbuf0, ssem0)
        widen(pbuf0, ebuf0, CH)
        drain(scatter(ibuf0, ebuf0))

        @pl.when(wid == NW - 1)
        def _():
            row0 = start + LAST_NCH * CH
            pltpu.sync_copy(dst_hbm.at[pl.ds(row0, 8), pl.ds(1, 1), :],
                            ibuf1.at[pl.ds(0, 8), :, :])
            pltpu.sync_copy(eblk_hbm.at[pl.ds(row0, 8), :, :],
                            pbuf1.at[pl.ds(0, 8), :, :])
            widen(pbuf1, ebuf1, 8)
            for j in range(8):
                pltpu.sync_copy(ebuf1.at[pl.ds(j * L, L), :],
                                acc.at[ibuf1.at[j, 0]], add=True)

        plsc.subcore_barrier()
        pltpu.sync_copy(acc.at[pl.ds(s * VS, VS), :],
                        part_hbm.at[pl.ds(c * NP + s * VS, VS), :])

    return run(dst2d, eblk, zeros)


PW = NP * DP // 128
BR = PW // 2


VB = NP // 128
VBB = VB // 2


def _combine_body(p_ref, b_ref, o_ref):
    q = p_ref[0] + p_ref[1]
    b = b_ref[...]
    y = pltpu.roll(q * b, 1, 1)
    l = lax.broadcasted_iota(jnp.int32, (BR, 128), 1) % DP
    w = jnp.where(l == 0, b, jnp.where(l <= D_E, y, 0.0))
    o_ref[...] = (w.reshape(VBB, DP, 16, DP)
                  .transpose(0, 3, 1, 2)
                  .reshape(VBB, DP, 128))


def _tc_combine(partials, bx):
    return pl.pallas_call(
        _combine_body,
        grid=(2,),
        in_specs=[
            pl.BlockSpec((2, BR, 128), lambda i: (0, i, 0)),
            pl.BlockSpec((BR, 128), lambda i: (i, 0)),
        ],
        out_specs=pl.BlockSpec((VBB, DP, 128), lambda i: (i, 0, 0)),
        out_shape=jax.ShapeDtypeStruct((VB, DP, 128), jnp.float32),
    )(partials, bx)


def kernel(vertex_attr, edgeij_pair, edge_attr, g, batch):
    dst2d = edgeij_pair.reshape(2, ROWS, L).transpose(1, 0, 2)
    eblk = edge_attr.reshape(ROWS, L, D_E).transpose(0, 2, 1)
    zeros = jnp.zeros((NP, DP), jnp.float32)
    part = _sc_scatter_partials(dst2d, eblk, zeros)
    p3 = part.reshape(NC, PW, 128)
    bpad = jnp.pad(vertex_attr[:, :1], ((0, NP - N), (0, 0)))
    bx = jnp.broadcast_to(bpad, (NP, DP)).reshape(PW, 128)
    ot = _tc_combine(p3, bx)
    return ot.transpose(1, 0, 2).reshape(DP, NP)[:1 + D_E, :N].T
